# baseline (speedup 1.0000x reference)
import functools

import jax
import jax.ops
import jax.numpy as jnp
import numpy as np
from jax import lax
from jax.experimental import pallas as pl
from jax.experimental.pallas import tpu as pltpu
from jax.experimental.pallas import tpu_sc as plsc

_NC = 2
_NS = 16
_LANES = 16

_TAU = 0.5
_ITERS = 10

_SB_MM = 512
_SB_RF = 2048


def _ceil_to(x, m):
    return (x + m - 1) // m * m


def _pick_zrows(q, cap):
    for z in range(min(cap, q) // 16 * 16, 15, -16):
        if q % z == 0:
            return z
    return 16


def _make_spmm(n, e_pad):
    sb, kb = _SB_MM, _SB_MM // 128
    nb = e_pad // sb
    q = _ceil_to(-(-n // _NS), 16)
    n_pad = q * _NS
    q_last = n - (_NS - 1) * q
    assert q_last > 0 and q_last % 8 == 0
    zrows = _pick_zrows(q, sb)
    assert q % zrows == 0 and zrows <= sb
    nz = q // zrows

    mesh = plsc.VectorSubcoreMesh(core_axis_name="c", subcore_axis_name="s",
                                  num_cores=_NC, num_subcores=_NS)

    @functools.partial(
        pl.kernel,
        mesh=mesh,
        out_type=(
            jax.ShapeDtypeStruct((n, 32), jnp.float32),
            jax.ShapeDtypeStruct((n, 32), jnp.float32),
        ),
        scratch_types=[
            pltpu.VMEM_SHARED((n_pad, 32), jnp.float32),
            pltpu.VMEM((sb, 32), jnp.float32),
            pltpu.VMEM((kb, 128), jnp.int32),
            pltpu.VMEM((kb, 128), jnp.int32),
            pltpu.VMEM((sb,), jnp.float32),
            pltpu.SemaphoreType.DMA,
        ],
        compiler_params=pltpu.CompilerParams(use_tc_tiling_on_sc=False,
                                             needs_layout_passes=False),
    )
    def spmm(x0_hbm, x1_hbm, col_hbm, row_hbm, val_hbm, out0_hbm, out1_hbm,
             acc_sh, rows_v, colb, rowb, valb, gsem):
        cid = lax.axis_index("c")
        sid = lax.axis_index("s")
        zero16 = jnp.zeros((_LANES,), jnp.float32)

        def body(x_hbm, out_hbm):
            def zrow(i, _):
                rows_v[i, pl.ds(0, 16)] = zero16
                rows_v[i, pl.ds(16, 16)] = zero16
                return 0
            lax.fori_loop(0, zrows, zrow, 0)
            for k in range(nz):
                pltpu.sync_copy(
                    rows_v.at[pl.ds(0, zrows)],
                    acc_sh.at[pl.ds(sid * q + k * zrows, zrows)])
            plsc.subcore_barrier()

            lo = sid * q
            hi = lo + q
            def super_body(b, _):
                rbase = b * kb
                ebase = rbase * 128
                pltpu.sync_copy(col_hbm.at[pl.ds(rbase, kb)], colb)
                pltpu.sync_copy(row_hbm.at[pl.ds(rbase, kb)], rowb)
                pltpu.sync_copy(val_hbm.at[pl.ds(ebase, sb)], valb)
                for j in range(kb):
                    pltpu.sync_copy(x_hbm.at[colb.at[j]],
                                    rows_v.at[pl.ds(j * 128, 128)])

                def mul(c, _):
                    jr = c // 8
                    off = (c % 8) * 16
                    row16 = rowb[jr, pl.ds(off, 16)]
                    owned = (row16 >= lo) & (row16 < hi)
                    dummy = lo + (row16 & 0x3FF)
                    rowb[jr, pl.ds(off, 16)] = jnp.where(owned, row16, dummy)
                    val16 = jnp.where(owned, valb[pl.ds(c * 16, 16)], 0.0)
                    base = c * 16
                    for l in range(16):
                        v16 = jnp.full((_LANES,), val16[l], jnp.float32)
                        rows_v[base + l, pl.ds(0, 16)] = (
                            rows_v[base + l, pl.ds(0, 16)] * v16)
                        rows_v[base + l, pl.ds(16, 16)] = (
                            rows_v[base + l, pl.ds(16, 16)] * v16)
                    return 0
                lax.fori_loop(0, sb // 16, mul, 0)

                for j in range(kb):
                    pltpu.sync_copy(
                        rows_v.at[pl.ds(j * 128, 128)],
                        acc_sh.at[rowb.at[j]], add=True)
                return 0
            lax.fori_loop(0, nb, super_body, 0)
            plsc.subcore_barrier()

            @pl.when(sid < _NS - 1)
            def _():
                pltpu.sync_copy(acc_sh.at[pl.ds(sid * q, q)],
                                out_hbm.at[pl.ds(sid * q, q)])

            @pl.when(sid == _NS - 1)
            def _():
                pltpu.sync_copy(
                    acc_sh.at[pl.ds((_NS - 1) * q, q_last)],
                    out_hbm.at[pl.ds((_NS - 1) * q, q_last)])

        @pl.when(cid == 0)
        def _():
            body(x0_hbm, out0_hbm)

        @pl.when(cid == 1)
        def _():
            body(x1_hbm, out1_hbm)

    return spmm


def _make_refine(n, e_pad):
    sb, kb = _SB_RF, _SB_RF // 128
    nb = e_pad // (_NS * sb)
    q = _ceil_to(-(-n // _NS), 16)
    n_pad = q * _NS
    q_last = n - (_NS - 1) * q
    assert q_last > 0 and q_last % 8 == 0
    zrows = _pick_zrows(q, 1024)
    nz = q // zrows
    assert q % zrows == 0 and q % 16 == 0 and zrows % 16 == 0

    mesh = plsc.VectorSubcoreMesh(core_axis_name="c", subcore_axis_name="s",
                                  num_cores=_NC, num_subcores=_NS)

    @functools.partial(
        pl.kernel,
        mesh=mesh,
        out_type=jax.ShapeDtypeStruct((n,), jnp.float32),
        scratch_types=[
            pltpu.VMEM_SHARED((n_pad,), jnp.float32),
            pltpu.VMEM_SHARED((n_pad,), jnp.float32),
            pltpu.VMEM_SHARED((_NS * _LANES,), jnp.float32),
            pltpu.VMEM((n_pad,), jnp.float32),
            pltpu.VMEM((q,), jnp.float32),
            pltpu.VMEM((kb, 128), jnp.int32),
            pltpu.VMEM((kb, 128), jnp.int32),
            pltpu.VMEM((sb,), jnp.float32),
            pltpu.VMEM((sb,), jnp.float32),
            pltpu.VMEM((zrows,), jnp.float32),
            pltpu.VMEM((_LANES,), jnp.float32),
            pltpu.VMEM((_NS * _LANES,), jnp.float32),
        ],
        compiler_params=pltpu.CompilerParams(use_tc_tiling_on_sc=False,
                                             needs_layout_passes=False),
    )
    def refine(col_hbm, row_hbm, val_hbm, v0_hbm, vout_hbm,
               v_sh, acc_sh, ss_sh, vbuf, ubuf, colb, rowb, valb, prodb,
               zbuf, ssp, ssb):
        cid = lax.axis_index("c")
        sid = lax.axis_index("s")
        zero16 = jnp.zeros((_LANES,), jnp.float32)

        @pl.when(cid == 0)
        def _():
            def zb(i, _):
                zbuf[pl.ds(i * 16, 16)] = zero16
                return 0
            lax.fori_loop(0, zrows // 16, zb, 0)

            @pl.when(sid < _NS - 1)
            def _():
                pltpu.sync_copy(v0_hbm.at[pl.ds(sid * q, q)],
                                v_sh.at[pl.ds(sid * q, q)])

            @pl.when(sid == _NS - 1)
            def _():
                pltpu.sync_copy(v0_hbm.at[pl.ds((_NS - 1) * q, q_last)],
                                v_sh.at[pl.ds((_NS - 1) * q, q_last)])
                if n_pad > n:
                    pltpu.sync_copy(zbuf.at[pl.ds(0, n_pad - n)],
                                    v_sh.at[pl.ds(n, n_pad - n)])
            plsc.subcore_barrier()

            def iteration(it, _):
                for k in range(nz):
                    pltpu.sync_copy(
                        zbuf, acc_sh.at[pl.ds(sid * q + k * zrows, zrows)])
                pltpu.sync_copy(v_sh, vbuf)
                plsc.subcore_barrier()

                def super_body(b, _):
                    rbase = (sid * nb + b) * kb
                    ebase = rbase * 128
                    pltpu.sync_copy(col_hbm.at[pl.ds(rbase, kb)], colb)
                    pltpu.sync_copy(row_hbm.at[pl.ds(rbase, kb)], rowb)
                    pltpu.sync_copy(val_hbm.at[pl.ds(ebase, sb)], valb)

                    def chunk(c, _):
                        jr = c // 8
                        off = (c % 8) * 16
                        col16 = colb[jr, pl.ds(off, 16)]
                        g16 = plsc.load_gather(vbuf, [col16])
                        prodb[pl.ds(c * 16, 16)] = (
                            g16 * valb[pl.ds(c * 16, 16)])
                        return 0
                    lax.fori_loop(0, sb // 16, chunk, 0)

                    for j in range(kb):
                        pltpu.sync_copy(
                            prodb.at[pl.ds(j * 128, 128)],
                            acc_sh.at[rowb.at[j]], add=True)
                    return 0
                lax.fori_loop(0, nb, super_body, 0)
                plsc.subcore_barrier()

                pltpu.sync_copy(acc_sh.at[pl.ds(sid * q, q)], ubuf)

                def thr(k, ss):
                    a = ubuf[pl.ds(k * 16, 16)]
                    u = a - _TAU * jnp.sign(a)
                    ubuf[pl.ds(k * 16, 16)] = u
                    return ss + u * u
                ss16 = lax.fori_loop(0, q // 16, thr, zero16)
                ssp[...] = ss16
                pltpu.sync_copy(ssp, ss_sh.at[pl.ds(sid * _LANES, _LANES)])
                plsc.subcore_barrier()

                pltpu.sync_copy(ss_sh, ssb)
                tot = zero16
                for t in range(_NS):
                    tot = tot + ssb[pl.ds(t * _LANES, _LANES)]
                ss = jnp.full((_LANES,), jnp.sum(tot), jnp.float32)
                x = jnp.maximum(ss, 1e-30)
                i = lax.bitcast_convert_type(x, jnp.int32)
                i = 0x5F3759DF - lax.shift_right_logical(i, 1)
                y = lax.bitcast_convert_type(i, jnp.float32)
                for _nw in range(4):
                    y = y * (1.5 - 0.5 * x * y * y)
                norm = x * y
                inv = 1.0 / jnp.maximum(norm, 1e-12)

                def scl(k, _):
                    ubuf[pl.ds(k * 16, 16)] = ubuf[pl.ds(k * 16, 16)] * inv
                    return 0
                lax.fori_loop(0, q // 16, scl, 0)
                pltpu.sync_copy(ubuf, v_sh.at[pl.ds(sid * q, q)])
                plsc.subcore_barrier()
                return 0

            lax.fori_loop(0, _ITERS, iteration, 0)

            @pl.when(sid < _NS - 1)
            def _():
                pltpu.sync_copy(ubuf, vout_hbm.at[pl.ds(sid * q, q)])

            @pl.when(sid == _NS - 1)
            def _():
                pltpu.sync_copy(ubuf.at[pl.ds(0, q_last)],
                                vout_hbm.at[pl.ds((_NS - 1) * q, q_last)])

    return refine


def _pad_edges(indices, values, n, sb):
    e = values.shape[0]
    e_pad = _ceil_to(e, _NS * sb)
    p = e_pad - e
    row = indices[0]
    col = indices[1]
    if p:
        fill = (jnp.arange(p, dtype=jnp.int32) * 97) % n
        row = jnp.concatenate([row, fill])
        col = jnp.concatenate([col, fill])
        values = jnp.concatenate([values, jnp.zeros((p,), values.dtype)])
    row2 = row.reshape(e_pad // 128, 128)
    col2 = col.reshape(e_pad // 128, 128)
    return col2, row2, values, e_pad


def kernel(A_indices, A_values, L_indices, L_values, embed, W1, W2, w_s):
    n = embed.shape[0]

    colA, rowA, valA, ea_pad = _pad_edges(A_indices, A_values, n, _SB_MM)

    spmm = _make_spmm(n, ea_pad)

    x = embed
    ax0, ax1 = spmm(x[:, :32], x[:, 32:], colA, rowA, valA)
    h = jnp.concatenate([ax0, ax1], axis=1) @ W1.T
    x = x + jnp.maximum(h, 0.0)
    ax0, ax1 = spmm(x[:, :32], x[:, 32:], colA, rowA, valA)
    h = jnp.concatenate([ax0, ax1], axis=1) @ W2.T
    x = x + jnp.maximum(h, 0.0)
    v = x @ w_s.T
    for _ in range(_ITERS):
        g = L_values[:, None] * jnp.take(v, L_indices[1], axis=0)
        v = jax.ops.segment_sum(g, L_indices[0], num_segments=n)
        v = v - _TAU * jnp.sign(v)
        nrm = jnp.linalg.norm(v, axis=0, keepdims=True)
        v = v / jnp.maximum(nrm, 1e-12)
    return jnp.squeeze(v, axis=-1)
